# baseline (speedup 1.0000x reference)
import functools

import jax
import jax.numpy as jnp
from jax.experimental import pallas as pl
from jax.experimental.pallas import tpu as pltpu

DQ = 128
DM = 512


def _tc_full(idx_ref, ct_ref, p_ref, sw_ref, sb_ref, w_ref, b_ref,
             o_ref, buf_ref, sem_ref, *, h, w, ch):
    nc = ct_ref.shape[0]
    nchunk = h // ch
    wc = w_ref[0:DQ, :]
    wp = w_ref[DQ:2 * DQ, :]
    ws = w_ref[2 * DQ:3 * DQ, :]
    size_e = float(h) * sw_ref[0:1, :] + float(w) * sw_ref[1:2, :] + sb_ref[0:1, :]
    const = jnp.dot(size_e, ws, preferred_element_type=jnp.float32) + b_ref[0:1, :]
    zt = jnp.dot(ct_ref[...], wc, preferred_element_type=jnp.float32)
    zt = jnp.concatenate([zt, jnp.zeros((DQ - nc, DM), jnp.float32)], axis=0)
    dn = (((2,), (0,)), ((), ()))
    for i in range(nchunk):
        b = i % 2
        idx = idx_ref[i * ch:(i + 1) * ch, :]
        lanes = jax.lax.broadcasted_iota(jnp.int32, (ch, w, DQ), 2)
        oh = (lanes == idx[:, :, None]).astype(jnp.float32)
        acc = jax.lax.dot_general(oh, zt, dn, preferred_element_type=jnp.float32)
        pos = p_ref[i * ch:(i + 1) * ch, :, :]
        acc = acc + jax.lax.dot_general(pos, wp, dn,
                                        preferred_element_type=jnp.float32)
        if i >= 2:
            pltpu.make_async_copy(
                buf_ref.at[b], o_ref.at[pl.ds((i - 2) * ch, ch)], sem_ref.at[b]
            ).wait()
        buf_ref[b] = acc + const.reshape(1, 1, DM)
        pltpu.make_async_copy(
            buf_ref.at[b], o_ref.at[pl.ds(i * ch, ch)], sem_ref.at[b]
        ).start()
    for i in range(max(nchunk - 2, 0), nchunk):
        b = i % 2
        pltpu.make_async_copy(
            buf_ref.at[b], o_ref.at[pl.ds(i * ch, ch)], sem_ref.at[b]
        ).wait()


def kernel(grid, color_table, pos_emb, size_W, size_b, combine_W, combine_b):
    h, w = grid.shape
    ch = 5
    return pl.pallas_call(
        functools.partial(_tc_full, h=h, w=w, ch=ch),
        out_shape=jax.ShapeDtypeStruct((h, w, DM), jnp.float32),
        out_specs=pl.BlockSpec(memory_space=pl.ANY),
        scratch_shapes=[
            pltpu.VMEM((2, ch, w, DM), jnp.float32),
            pltpu.SemaphoreType.DMA((2,)),
        ],
    )(
        grid.astype(jnp.int32),
        color_table,
        pos_emb[:h, :w],
        size_W,
        size_b.reshape(1, DQ),
        combine_W,
        combine_b.reshape(1, DM),
    )


# speedup vs baseline: 1.3174x; 1.3174x over previous
import functools

import jax
import jax.numpy as jnp
from jax.experimental import pallas as pl
from jax.experimental.pallas import tpu as pltpu

DQ = 128
DM = 512


def _tc_full(idx_ref, ct_ref, p_ref, sw_ref, sb_ref, w_ref, b_ref,
             o_ref, *, h, w):
    nc = ct_ref.shape[0]
    wc = w_ref[0:DQ, :]
    wp = w_ref[DQ:2 * DQ, :]
    ws = w_ref[2 * DQ:3 * DQ, :]
    size_e = float(h) * sw_ref[0:1, :] + float(w) * sw_ref[1:2, :] + sb_ref[0:1, :]
    const = jnp.dot(size_e, ws, preferred_element_type=jnp.float32) + b_ref[0:1, :]
    zt = jnp.dot(ct_ref[...], wc, preferred_element_type=jnp.float32)
    zt = jnp.concatenate(
        [zt, jnp.zeros((DQ - nc - 1, DM), jnp.float32), const], axis=0)
    lanes = jax.lax.broadcasted_iota(jnp.int32, (h, w, DQ), 2)
    oh = ((lanes == idx_ref[...][:, :, None]) | (lanes == DQ - 1)
          ).astype(jnp.float32)
    dn = (((2,), (0,)), ((), ()))
    acc = jax.lax.dot_general(oh, zt, dn, preferred_element_type=jnp.float32)
    acc = acc + jax.lax.dot_general(p_ref[...], wp, dn,
                                    preferred_element_type=jnp.float32)
    o_ref[...] = acc


def kernel(grid, color_table, pos_emb, size_W, size_b, combine_W, combine_b):
    h, w = grid.shape
    return pl.pallas_call(
        functools.partial(_tc_full, h=h, w=w),
        out_shape=jax.ShapeDtypeStruct((h, w, DM), jnp.float32),
    )(
        grid.astype(jnp.int32),
        color_table,
        pos_emb[:h, :w],
        size_W,
        size_b.reshape(1, DQ),
        combine_W,
        combine_b.reshape(1, DM),
    )


# speedup vs baseline: 1.3503x; 1.0250x over previous
import functools

import jax
import jax.numpy as jnp
from jax.experimental import pallas as pl
from jax.experimental.pallas import tpu as pltpu

DQ = 128
DM = 512


def _tc_full(idx_ref, ct_ref, p_ref, sw_ref, sb_ref, w_ref, b_ref,
             o_ref, *, h, w):
    nc = ct_ref.shape[0]
    wc = w_ref[0:DQ, :]
    wp = w_ref[DQ:2 * DQ, :]
    ws = w_ref[2 * DQ:3 * DQ, :]
    size_e = float(h) * sw_ref[0:1, :] + float(w) * sw_ref[1:2, :] + sb_ref[0:1, :]
    const = jnp.dot(size_e, ws, preferred_element_type=jnp.float32) + b_ref[0:1, :]
    zt = jnp.dot(ct_ref[...], wc, preferred_element_type=jnp.float32)
    zt = jnp.concatenate(
        [zt, jnp.zeros((DQ - nc - 1, DM), jnp.float32), const], axis=0)
    lanes = jax.lax.broadcasted_iota(jnp.int32, (h, w, DQ), 2)
    oh = ((lanes == idx_ref[...][:, :, None]) | (lanes == DQ - 1)
          ).astype(jnp.float32)
    dn = (((2,), (0,)), ((), ()))
    lhs = jnp.concatenate([oh, p_ref[...]], axis=2)
    rhs = jnp.concatenate([zt, wp], axis=0)
    o_ref[...] = jax.lax.dot_general(lhs, rhs, dn,
                                     preferred_element_type=jnp.float32)


def kernel(grid, color_table, pos_emb, size_W, size_b, combine_W, combine_b):
    h, w = grid.shape
    return pl.pallas_call(
        functools.partial(_tc_full, h=h, w=w),
        out_shape=jax.ShapeDtypeStruct((h, w, DM), jnp.float32),
    )(
        grid.astype(jnp.int32),
        color_table,
        pos_emb[:h, :w],
        size_W,
        size_b.reshape(1, DQ),
        combine_W,
        combine_b.reshape(1, DM),
    )
